# Initial kernel scaffold; baseline (speedup 1.0000x reference)
import jax
import jax.numpy as jnp
from jax.experimental import pallas as pl


def kernel(nodes, edge_index, active_nodes):
    raise NotImplementedError("write your pallas kernel here")


# speedup vs baseline: 12.2768x; 12.2768x over previous
import functools

import jax
import jax.numpy as jnp
from jax import lax
from jax.experimental import pallas as pl
from jax.experimental.pallas import tpu as pltpu
from jax.experimental.pallas import tpu_sc as plsc

_N = 10000
_D = 128
_E = 320000
_NW = 32
_EPW = _E // _NW
_LANES = 16

_sc_mesh = plsc.VectorSubcoreMesh(core_axis_name="c", subcore_axis_name="s")


@functools.partial(
    pl.kernel,
    out_type=jax.ShapeDtypeStruct((2 * _NW * _N,), jnp.float32),
    mesh=_sc_mesh,
    scratch_types=[
        pltpu.VMEM((_EPW,), jnp.int32),
        pltpu.VMEM((_EPW,), jnp.int32),
        pltpu.VMEM((2 * _N,), jnp.float32),
    ],
    compiler_params=pltpu.CompilerParams(needs_layout_passes=False),
)
def _sc_hist(edges_hbm, out_hbm, sidx, ridx, hist):
    wid = lax.axis_index("s") * 2 + lax.axis_index("c")
    base = wid * _EPW
    pltpu.sync_copy(edges_hbm.at[pl.ds(base, _EPW)], sidx)
    pltpu.sync_copy(edges_hbm.at[pl.ds(_E + base, _EPW)], ridx)

    zero16 = jnp.zeros((_LANES,), jnp.float32)

    def zbody(i, c):
        hist[pl.ds(i * _LANES, _LANES)] = zero16
        return c

    lax.fori_loop(0, (2 * _N) // _LANES, zbody, 0)

    ones16 = jnp.ones((_LANES,), jnp.float32)
    noff = jnp.full((_LANES,), _N, jnp.int32)

    def body(i, c):
        s = sidx[pl.ds(i * _LANES, _LANES)]
        plsc.addupdate_scatter(hist, [s], ones16)
        r = ridx[pl.ds(i * _LANES, _LANES)]
        plsc.addupdate_scatter(hist, [r + noff], ones16)
        return c

    lax.fori_loop(0, _EPW // _LANES, body, 0)

    pltpu.sync_copy(hist.at[pl.ds(0, _N)], out_hbm.at[pl.ds(wid * _N, _N)])
    pltpu.sync_copy(hist.at[pl.ds(_N, _N)],
                    out_hbm.at[pl.ds((_NW + wid) * _N, _N)])


_BLK = 2000


def _tc_body(nodes_ref, part_ref, act_ref, out_ref, deg_scr):
    i = pl.program_id(0)

    @pl.when(i == 0)
    def _compute_deg():
        p = part_ref[...].reshape(2 * _NW, _N)
        jm = lax.broadcasted_iota(jnp.int32, (2 * _NW, _D), 0)
        lm = lax.broadcasted_iota(jnp.int32, (2 * _NW, _D), 1)
        w = ((jm // _NW) == lm).astype(jnp.float32)
        deg_scr[...] = lax.dot_general(
            p, w, (((0,), (0,)), ((), ())),
            preferred_element_type=jnp.float32)

    res = deg_scr[pl.ds(i * _BLK, _BLK), :] * act_ref[...]
    col = lax.broadcasted_iota(jnp.int32, (_BLK, _D), 1)
    out_ref[...] = jnp.where(col < 2, res, nodes_ref[...])


_tc_inject = pl.pallas_call(
    _tc_body,
    grid=(_N // _BLK,),
    in_specs=[
        pl.BlockSpec((_BLK, _D), lambda i: (i, 0)),
        pl.BlockSpec((2, _NW, _N), lambda i: (0, 0, 0)),
        pl.BlockSpec((_BLK, 1), lambda i: (i, 0)),
    ],
    out_specs=pl.BlockSpec((_BLK, _D), lambda i: (i, 0)),
    out_shape=jax.ShapeDtypeStruct((_N, _D), jnp.float32),
    scratch_shapes=[pltpu.VMEM((_N, _D), jnp.float32)],
)


def kernel(nodes, edge_index, active_nodes):
    edge_flat = edge_index.reshape(-1)
    partials = _sc_hist(edge_flat).reshape(2, _NW, _N)
    act2 = active_nodes.reshape(-1, 1)
    return _tc_inject(nodes, partials, act2)


# speedup vs baseline: 19.7096x; 1.6054x over previous
import functools

import jax
import jax.numpy as jnp
from jax import lax
from jax.experimental import pallas as pl
from jax.experimental.pallas import tpu as pltpu
from jax.experimental.pallas import tpu_sc as plsc

_N = 10000
_D = 128
_E = 320000
_NW = 32
_EPW = _E // _NW
_LANES = 16
_NP = 10240
_EPW0 = 9984
_EXTRA = _E - _NW * _EPW0
_EBUF = _EPW0 + _EXTRA

_sc_mesh = plsc.VectorSubcoreMesh(core_axis_name="c", subcore_axis_name="s")


@functools.partial(
    pl.kernel,
    out_type=jax.ShapeDtypeStruct((_NW * 2 * _NP,), jnp.float32),
    mesh=_sc_mesh,
    scratch_types=[
        pltpu.VMEM((2, _EBUF), jnp.int32),
        pltpu.VMEM((2 * _NP,), jnp.float32),
        pltpu.SemaphoreType.DMA,
        pltpu.SemaphoreType.DMA,
    ],
    compiler_params=pltpu.CompilerParams(needs_layout_passes=False),
)
def _sc_hist(edges_hbm, out_hbm, eidx, hist, sem_s, sem_r):
    wid = lax.axis_index("s") * 2 + lax.axis_index("c")
    base = wid * _EPW0
    cp_s = pltpu.async_copy(edges_hbm.at[:, pl.ds(base, _EPW0)],
                            eidx.at[:, pl.ds(0, _EPW0)], sem_s)

    @pl.when(wid == _NW - 1)
    def _extra_start():
        pltpu.async_copy(edges_hbm.at[:, pl.ds(_NW * _EPW0, _EXTRA)],
                         eidx.at[:, pl.ds(_EPW0, _EXTRA)], sem_r)

    zero16 = jnp.zeros((_LANES,), jnp.float32)

    @plsc.parallel_loop(0, (2 * _NP) // _LANES, unroll=4)
    def _zero(i):
        hist[pl.ds(i * _LANES, _LANES)] = zero16

    cp_s.wait()

    @pl.when(wid == _NW - 1)
    def _extra_wait():
        pltpu.make_async_copy(
            edges_hbm.at[:, pl.ds(_NW * _EPW0, _EXTRA)],
            eidx.at[:, pl.ds(_EPW0, _EXTRA)], sem_r).wait()

    ones16 = jnp.ones((_LANES,), jnp.float32)
    noff = jnp.full((_LANES,), _NP, jnp.int32)

    @plsc.parallel_loop(0, _EPW0 // _LANES, unroll=16)
    def _scatter(i):
        s = eidx[0, pl.ds(i * _LANES, _LANES)]
        plsc.addupdate_scatter(hist, [s], ones16)
        r = eidx[1, pl.ds(i * _LANES, _LANES)]
        plsc.addupdate_scatter(hist, [r + noff], ones16)

    @pl.when(wid == _NW - 1)
    def _extra_scatter():
        @plsc.parallel_loop(0, _EXTRA // _LANES, unroll=2)
        def _scatter2(i):
            s = eidx[0, pl.ds(_EPW0 + i * _LANES, _LANES)]
            plsc.addupdate_scatter(hist, [s], ones16)
            r = eidx[1, pl.ds(_EPW0 + i * _LANES, _LANES)]
            plsc.addupdate_scatter(hist, [r + noff], ones16)

    pltpu.sync_copy(hist, out_hbm.at[pl.ds(wid * 2 * _NP, 2 * _NP)])


_BLK = 2000
_R = 2 * _NW


def _tc_body(nodes_ref, part_hbm, act_ref, out_ref, deg_scr, pbuf, sem):
    i = pl.program_id(0)

    @pl.when(i == 0)
    def _compute_deg():
        for j in range(_R):
            pltpu.make_async_copy(
                part_hbm.at[pl.ds(j * _NP, _NP)], pbuf.at[j], sem).start()
        for j in range(_R):
            pltpu.make_async_copy(
                part_hbm.at[pl.ds(j * _NP, _NP)], pbuf.at[j], sem).wait()
        act_full = jnp.concatenate(
            [act_ref[...], jnp.zeros((_NP - _N,), jnp.float32)])
        act_b = jnp.broadcast_to(act_full[None, :], (_R, _NP))
        p = pbuf[...] * act_b
        jm = lax.broadcasted_iota(jnp.int32, (_R, _D), 0)
        lm = lax.broadcasted_iota(jnp.int32, (_R, _D), 1)
        w = ((jm % 2) == lm).astype(jnp.float32)
        full = lax.dot_general(
            p, w, (((0,), (0,)), ((), ())),
            preferred_element_type=jnp.float32)
        deg_scr[...] = full[0:_N, :]

    res = deg_scr[pl.ds(i * _BLK, _BLK), :]
    col = lax.broadcasted_iota(jnp.int32, (_BLK, _D), 1)
    out_ref[...] = jnp.where(col < 2, res, nodes_ref[...])


_tc_inject = pl.pallas_call(
    _tc_body,
    grid=(_N // _BLK,),
    in_specs=[
        pl.BlockSpec((_BLK, _D), lambda i: (i, 0)),
        pl.BlockSpec(memory_space=pltpu.MemorySpace.HBM),
        pl.BlockSpec((_N,), lambda i: (0,)),
    ],
    out_specs=pl.BlockSpec((_BLK, _D), lambda i: (i, 0)),
    out_shape=jax.ShapeDtypeStruct((_N, _D), jnp.float32),
    scratch_shapes=[
        pltpu.VMEM((_N, _D), jnp.float32),
        pltpu.VMEM((_R, _NP), jnp.float32),
        pltpu.SemaphoreType.DMA,
    ],
)


def kernel(nodes, edge_index, active_nodes):
    partials = _sc_hist(edge_index)
    return _tc_inject(nodes, partials, active_nodes)
